# baseline (speedup 1.0000x reference)
import functools

import jax
import jax.numpy as jnp
import numpy as np
from jax import lax
from jax.experimental import pallas as pl
from jax.experimental.pallas import tpu as pltpu

SEL = 4096
TJ = 8192
LANES = 128
SUB = 8
MININT = np.int32(-2147483648)
TINY = np.float32(np.finfo(np.float32).tiny)
KS1 = np.int32(42)
KS2 = np.int32(np.uint32(0x1BD11BDA) ^ np.uint32(42))
ROT = (13, 15, 26, 6, 17, 29, 16, 24)


def _rotl(x, d):
    return (x << d) | lax.shift_right_logical(x, 32 - d)


def _threefry2x32(x0, x1):
    ks = (np.int32(0), KS1, KS2)
    x1 = x1 + ks[1]
    for group in range(5):
        for r in range(4):
            d = ROT[(group % 2) * 4 + r]
            x0 = x0 + x1
            x1 = _rotl(x1, d)
            x1 = x1 ^ x0
        x0 = x0 + ks[(group + 1) % 3]
        x1 = x1 + np.int32(int(ks[(group + 2) % 3]) + group + 1)
    return x0 ^ x1


def _body(nn, n_tiles, rows_half, pr_ref, tg_ref, out_ref,
          accs_ref, accw_ref, w_ref, l_ref):
    h = pl.program_id(0)
    a = pl.program_id(1)
    iblocks = rows_half // SUB
    rows_per_tile = TJ // LANES

    p = jnp.clip(pr_ref[0], 1e-07, 1.0 - 1e-07)
    t = tg_ref[0].astype(jnp.float32)
    w64 = jnp.abs(p - t)
    w_ref[...] = w64
    l_ref[...] = jnp.log(w64)

    @pl.when(a == 0)
    def _init():
        accs_ref[...] = jnp.full(accs_ref.shape, -jnp.inf, jnp.float32)
        accw_ref[...] = jnp.zeros(accw_ref.shape, jnp.float32)

    lane = lax.broadcasted_iota(jnp.int32, (SUB, LANES), 1)
    sub = lax.broadcasted_iota(jnp.int32, (SUB, LANES), 0)
    mrow = nn // 8192

    def iblk_body(ib, _):
        ivec = h * rows_half + ib * SUB + sub
        m = ivec * mrow
        bhi = lax.shift_right_logical(m, 19)
        blo = m << 13
        blox = blo ^ MININT

        def r_body(r, car):
            accs, accw = car
            jfull = a * TJ + r * LANES + lane
            lo = blo + jfull
            carry = jnp.where((lo ^ MININT) < blox, np.int32(1), np.int32(0))
            hi = bhi + carry
            bits = _threefry2x32(hi, lo)
            sh = lax.shift_right_logical(bits, 9)
            f = lax.bitcast_convert_type(sh | np.int32(0x3F800000), jnp.float32)
            u = (f - 1.0) * (1.0 - TINY) + TINY
            u = jnp.maximum(TINY, u)
            g = -jnp.log(-jnp.log(u))
            lrow = jnp.broadcast_to(l_ref[pl.ds(r, 1), :], (SUB, LANES))
            wrow = jnp.broadcast_to(w_ref[pl.ds(r, 1), :], (SUB, LANES))
            s = g + lrow
            upd = s > accs
            accs = jnp.where(upd, s, accs)
            accw = jnp.where(upd, wrow, accw)
            return accs, accw

        accs, accw = lax.fori_loop(
            0, rows_per_tile, r_body, (accs_ref[ib], accw_ref[ib]))
        accs_ref[ib] = accs
        accw_ref[ib] = accw
        return 0

    lax.fori_loop(0, iblocks, iblk_body, 0)

    @pl.when(a == n_tiles - 1)
    def _finalize():
        def fin_body(ib, tot):
            accs = accs_ref[ib]
            accw = accw_ref[ib]
            best = jnp.max(accs, axis=1, keepdims=True)
            wb = jnp.max(jnp.where(accs == best, accw, -1.0),
                         axis=1, keepdims=True)
            return tot + jnp.sum(-jnp.log(1.0 - wb))

        total = lax.fori_loop(0, iblocks, fin_body, jnp.float32(0.0))
        out_ref[...] = jnp.full((1, 1, LANES), total, jnp.float32)


def kernel(prob, targets):
    rr, cc = prob.shape
    nn = rr * cc
    n_tiles = nn // TJ
    rows_half = SEL // 2
    pr3 = prob.reshape(n_tiles, TJ // LANES, LANES)
    tg3 = targets.reshape(n_tiles, TJ // LANES, LANES)

    out = pl.pallas_call(
        functools.partial(_body, nn, n_tiles, rows_half),
        grid=(2, n_tiles),
        in_specs=[
            pl.BlockSpec((1, TJ // LANES, LANES), lambda h, a: (a, 0, 0)),
            pl.BlockSpec((1, TJ // LANES, LANES), lambda h, a: (a, 0, 0)),
        ],
        out_specs=pl.BlockSpec((1, 1, LANES), lambda h, a: (h, 0, 0)),
        out_shape=jax.ShapeDtypeStruct((2, 1, LANES), jnp.float32),
        scratch_shapes=[
            pltpu.VMEM((rows_half // SUB, SUB, LANES), jnp.float32),
            pltpu.VMEM((rows_half // SUB, SUB, LANES), jnp.float32),
            pltpu.VMEM((TJ // LANES, LANES), jnp.float32),
            pltpu.VMEM((TJ // LANES, LANES), jnp.float32),
        ],
        compiler_params=pltpu.CompilerParams(
            dimension_semantics=("parallel", "arbitrary"),
        ),
    )(pr3, tg3)

    return (out[0, 0, 0] + out[1, 0, 0]) / SEL


# speedup vs baseline: 5.0702x; 5.0702x over previous
import functools

import jax
import jax.numpy as jnp
import numpy as np
from jax import lax
from jax.experimental import pallas as pl
from jax.experimental.pallas import tpu as pltpu

SEL = 4096
TJ = 8192
WIDE = 1024
LANES = 128
SUB = 8
MININT = np.int32(-2147483648)
TINY = np.float32(np.finfo(np.float32).tiny)
KS1 = np.int32(42)
KS2 = np.int32(np.uint32(0x1BD11BDA) ^ np.uint32(42))
ROT = (13, 15, 26, 6, 17, 29, 16, 24)


def _rotl(x, d):
    return (x << d) | lax.shift_right_logical(x, 32 - d)


def _threefry2x32(x0, x1):
    ks = (np.int32(0), KS1, KS2)
    x1 = x1 + ks[1]
    for group in range(5):
        for r in range(4):
            d = ROT[(group % 2) * 4 + r]
            x0 = x0 + x1
            x1 = _rotl(x1, d)
            x1 = x1 ^ x0
        x0 = x0 + ks[(group + 1) % 3]
        x1 = x1 + np.int32(int(ks[(group + 2) % 3]) + group + 1)
    return x0 ^ x1


def _body(nn, n_tiles, rows_half, pr_ref, tg_ref, out_ref,
          accs_ref, accw_ref, w_ref, l_ref):
    h = pl.program_id(0)
    a = pl.program_id(1)
    iblocks = rows_half // SUB
    wides_per_tile = TJ // WIDE

    p = jnp.clip(pr_ref[0], 1e-07, 1.0 - 1e-07)
    t = tg_ref[0].astype(jnp.float32)
    w8 = jnp.abs(p - t)
    w_ref[...] = w8
    l_ref[...] = jnp.log(w8)

    @pl.when(a == 0)
    def _init():
        accs_ref[...] = jnp.full(accs_ref.shape, -jnp.inf, jnp.float32)
        accw_ref[...] = jnp.zeros(accw_ref.shape, jnp.float32)

    lane = lax.broadcasted_iota(jnp.int32, (SUB, WIDE), 1)
    sub1 = lax.broadcasted_iota(jnp.int32, (SUB, 1), 0)
    mrow = nn // 8192

    def iblk_body(ib, _):
        ivec = h * rows_half + ib * SUB + sub1
        m = ivec * mrow
        bhi = lax.shift_right_logical(m, 19)
        blo = m << 13
        blox = blo ^ MININT

        def r_body(r, car):
            accs, accw = car
            jfull = a * TJ + r * WIDE + lane
            lo = blo + jfull
            carry = jnp.where((lo ^ MININT) < blox, np.int32(1), np.int32(0))
            hi = bhi + carry
            bits = _threefry2x32(hi, lo)
            sh = lax.shift_right_logical(bits, 9)
            f = lax.bitcast_convert_type(sh | np.int32(0x3F800000), jnp.float32)
            u = (f - 1.0) * (1.0 - TINY) + TINY
            u = jnp.maximum(TINY, u)
            g = -jnp.log(-jnp.log(u))
            lrow = jnp.broadcast_to(l_ref[pl.ds(r, 1), :], (SUB, WIDE))
            wrow = jnp.broadcast_to(w_ref[pl.ds(r, 1), :], (SUB, WIDE))
            s = g + lrow
            upd = s > accs
            accs = jnp.where(upd, s, accs)
            accw = jnp.where(upd, wrow, accw)
            return accs, accw

        accs, accw = lax.fori_loop(
            0, wides_per_tile, r_body, (accs_ref[ib], accw_ref[ib]))
        accs_ref[ib] = accs
        accw_ref[ib] = accw
        return 0

    lax.fori_loop(0, iblocks, iblk_body, 0)

    @pl.when(a == n_tiles - 1)
    def _finalize():
        def fin_body(ib, tot):
            accs = accs_ref[ib]
            accw = accw_ref[ib]
            best = jnp.max(accs, axis=1, keepdims=True)
            wb = jnp.max(jnp.where(accs == best, accw, -1.0),
                         axis=1, keepdims=True)
            return tot + jnp.sum(-jnp.log(1.0 - wb))

        total = lax.fori_loop(0, iblocks, fin_body, jnp.float32(0.0))
        out_ref[...] = jnp.full((1, 1, LANES), total, jnp.float32)


def kernel(prob, targets):
    rr, cc = prob.shape
    nn = rr * cc
    n_tiles = nn // TJ
    rows_half = SEL // 2
    pr3 = prob.reshape(n_tiles, TJ // WIDE, WIDE)
    tg3 = targets.reshape(n_tiles, TJ // WIDE, WIDE)

    out = pl.pallas_call(
        functools.partial(_body, nn, n_tiles, rows_half),
        grid=(2, n_tiles),
        in_specs=[
            pl.BlockSpec((1, TJ // WIDE, WIDE), lambda h, a: (a, 0, 0)),
            pl.BlockSpec((1, TJ // WIDE, WIDE), lambda h, a: (a, 0, 0)),
        ],
        out_specs=pl.BlockSpec((1, 1, LANES), lambda h, a: (h, 0, 0)),
        out_shape=jax.ShapeDtypeStruct((2, 1, LANES), jnp.float32),
        scratch_shapes=[
            pltpu.VMEM((rows_half // SUB, SUB, WIDE), jnp.float32),
            pltpu.VMEM((rows_half // SUB, SUB, WIDE), jnp.float32),
            pltpu.VMEM((TJ // WIDE, WIDE), jnp.float32),
            pltpu.VMEM((TJ // WIDE, WIDE), jnp.float32),
        ],
        compiler_params=pltpu.CompilerParams(
            dimension_semantics=("parallel", "arbitrary"),
        ),
    )(pr3, tg3)

    return (out[0, 0, 0] + out[1, 0, 0]) / SEL
